# baseline (speedup 1.0000x reference)
import jax
import jax.numpy as jnp
from jax.experimental import pallas as pl
from jax.experimental.pallas import tpu as pltpu

B = 128
D_MODEL = 768
VOCAB = 32000
E = 8
VBLK = 6400
QUART = VBLK // 4
V_TILES = VOCAB // VBLK


def _moe_body(p_ref, x_ref, w0_ref, w1_ref, w2_ref, w3_ref, b_ref, o_ref):
    e = pl.program_id(1)
    mask = (p_ref[:] % E) == e
    xb = x_ref[:].astype(jnp.bfloat16)
    first = e == 0
    for i, w_ref in enumerate((w0_ref, w1_ref, w2_ref, w3_ref)):
        acc = jax.lax.dot_general(
            xb, w_ref[0].astype(jnp.bfloat16),
            dimension_numbers=(((1,), (1,)), ((), ())),
            preferred_element_type=jnp.float32,
        )
        acc = acc + b_ref[0, :, i * QUART:(i + 1) * QUART]
        sl = (slice(None), slice(i * QUART, (i + 1) * QUART))

        @pl.when(first)
        def _(acc=acc, sl=sl):
            o_ref[sl] = jnp.where(mask, acc, jnp.zeros_like(acc))

        @pl.when(jnp.logical_not(first))
        def _(acc=acc, sl=sl):
            o_ref[sl] = jnp.where(mask, acc, o_ref[sl])


def kernel(x, pointer_addresses, W, b):
    p2d = pointer_addresses.reshape(B, 1).astype(jnp.int32)
    wspec = lambda i: pl.BlockSpec(
        (1, QUART, D_MODEL), lambda v, e, i=i: (e, 4 * v + i, 0))
    out = pl.pallas_call(
        _moe_body,
        grid=(V_TILES, E),
        in_specs=[
            pl.BlockSpec((B, 1), lambda v, e: (0, 0)),
            pl.BlockSpec((B, D_MODEL), lambda v, e: (0, 0)),
            wspec(0), wspec(1), wspec(2), wspec(3),
            pl.BlockSpec((1, 1, VBLK), lambda v, e: (e, 0, v)),
        ],
        out_specs=pl.BlockSpec((B, VBLK), lambda v, e: (0, v)),
        out_shape=jax.ShapeDtypeStruct((B, VOCAB), jnp.float32),
        compiler_params=pltpu.CompilerParams(
            dimension_semantics=("arbitrary", "arbitrary"),
        ),
    )(p2d, x, W, W, W, W, b.reshape(E, 1, VOCAB))
    return out


# speedup vs baseline: 1.0463x; 1.0463x over previous
import jax
import jax.numpy as jnp
from jax.experimental import pallas as pl
from jax.experimental.pallas import tpu as pltpu

B = 128
D_MODEL = 768
VOCAB = 32000
E = 8
VBLK = 6400
HALF = VBLK // 2
V_TILES = VOCAB // VBLK


def _moe_body(p_ref, x_ref, wa_ref, wb_ref, b_ref, o_ref):
    e = pl.program_id(1)
    mask = (p_ref[:] % E) == e
    xb = x_ref[:].astype(jnp.bfloat16)
    acc_a = jax.lax.dot_general(
        xb, wa_ref[0].astype(jnp.bfloat16),
        dimension_numbers=(((1,), (1,)), ((), ())),
        preferred_element_type=jnp.float32,
    )
    acc_b = jax.lax.dot_general(
        xb, wb_ref[0].astype(jnp.bfloat16),
        dimension_numbers=(((1,), (1,)), ((), ())),
        preferred_element_type=jnp.float32,
    )
    first = e == 0
    for i, acc in enumerate((acc_a, acc_b)):
        acc = acc + b_ref[0, :, i * HALF:(i + 1) * HALF]
        sl = (slice(None), slice(i * HALF, (i + 1) * HALF))

        @pl.when(first)
        def _(acc=acc, sl=sl):
            o_ref[sl] = jnp.where(mask, acc, jnp.zeros_like(acc))

        @pl.when(jnp.logical_not(first))
        def _(acc=acc, sl=sl):
            o_ref[sl] = jnp.where(mask, acc, o_ref[sl])


def kernel(x, pointer_addresses, W, b):
    p2d = pointer_addresses.reshape(B, 1).astype(jnp.int32)
    out = pl.pallas_call(
        _moe_body,
        grid=(V_TILES, E),
        in_specs=[
            pl.BlockSpec((B, 1), lambda v, e: (0, 0)),
            pl.BlockSpec((B, D_MODEL), lambda v, e: (0, 0)),
            pl.BlockSpec((1, HALF, D_MODEL), lambda v, e: (e, 2 * v, 0)),
            pl.BlockSpec((1, HALF, D_MODEL), lambda v, e: (e, 2 * v + 1, 0)),
            pl.BlockSpec((1, 1, VBLK), lambda v, e: (e, 0, v)),
        ],
        out_specs=pl.BlockSpec((B, VBLK), lambda v, e: (0, v)),
        out_shape=jax.ShapeDtypeStruct((B, VOCAB), jnp.float32),
        compiler_params=pltpu.CompilerParams(
            dimension_semantics=("arbitrary", "arbitrary"),
        ),
    )(p2d, x, W, W, b.reshape(E, 1, VOCAB))
    return out


# speedup vs baseline: 1.0768x; 1.0291x over previous
import jax
import jax.numpy as jnp
from jax.experimental import pallas as pl
from jax.experimental.pallas import tpu as pltpu

B = 128
D_MODEL = 768
VOCAB = 32000
E = 8
VBLK = 6400
HALF = VBLK // 2
V_TILES = VOCAB // VBLK


def _moe_body(p_ref, x_ref, wa_ref, wb_ref, b_ref, o_ref):
    e = pl.program_id(1)
    mask = (p_ref[:] % E) == e
    xb = x_ref[:].astype(jnp.bfloat16)
    acc_a = jax.lax.dot_general(
        xb, wa_ref[0].astype(jnp.bfloat16),
        dimension_numbers=(((1,), (1,)), ((), ())),
        preferred_element_type=jnp.float32,
    )
    acc_b = jax.lax.dot_general(
        xb, wb_ref[0].astype(jnp.bfloat16),
        dimension_numbers=(((1,), (1,)), ((), ())),
        preferred_element_type=jnp.float32,
    )
    acc = jnp.concatenate([acc_a, acc_b], axis=1) + b_ref[0]

    @pl.when(e == 0)
    def _():
        o_ref[:] = jnp.where(mask, acc, jnp.zeros_like(acc))

    @pl.when(e != 0)
    def _():
        o_ref[:] = jnp.where(mask, acc, o_ref[:])


def kernel(x, pointer_addresses, W, b):
    p2d = pointer_addresses.reshape(B, 1).astype(jnp.int32)
    out = pl.pallas_call(
        _moe_body,
        grid=(V_TILES, E),
        in_specs=[
            pl.BlockSpec((B, 1), lambda v, e: (0, 0)),
            pl.BlockSpec((B, D_MODEL), lambda v, e: (0, 0)),
            pl.BlockSpec((1, HALF, D_MODEL), lambda v, e: (e, 2 * v, 0)),
            pl.BlockSpec((1, HALF, D_MODEL), lambda v, e: (e, 2 * v + 1, 0)),
            pl.BlockSpec((1, 1, VBLK), lambda v, e: (e, 0, v)),
        ],
        out_specs=pl.BlockSpec((B, VBLK), lambda v, e: (0, v)),
        out_shape=jax.ShapeDtypeStruct((B, VOCAB), jnp.float32),
        compiler_params=pltpu.CompilerParams(
            dimension_semantics=("arbitrary", "arbitrary"),
        ),
    )(p2d, x, W, W, b.reshape(E, 1, VOCAB))
    return out


# speedup vs baseline: 1.0792x; 1.0022x over previous
import jax
import jax.numpy as jnp
from jax.experimental import pallas as pl
from jax.experimental.pallas import tpu as pltpu

B = 128
D_MODEL = 768
VOCAB = 32000
E = 8
VBLK = 6400
HALF = VBLK // 2
V_TILES = VOCAB // VBLK


def _moe_body(p_ref, x_ref, wa_ref, wb_ref, b_ref, o_ref):
    e = pl.program_id(1)
    mask = (p_ref[:] % E) == e
    xb = x_ref[:]
    acc_a = jax.lax.dot_general(
        xb, wa_ref[0],
        dimension_numbers=(((1,), (1,)), ((), ())),
        preferred_element_type=jnp.float32,
    )
    acc_b = jax.lax.dot_general(
        xb, wb_ref[0],
        dimension_numbers=(((1,), (1,)), ((), ())),
        preferred_element_type=jnp.float32,
    )
    acc = jnp.concatenate([acc_a, acc_b], axis=1) + b_ref[0]

    @pl.when(e == 0)
    def _():
        o_ref[:] = jnp.where(mask, acc, jnp.zeros_like(acc))

    @pl.when(e != 0)
    def _():
        o_ref[:] = jnp.where(mask, acc, o_ref[:])


def kernel(x, pointer_addresses, W, b):
    p2d = pointer_addresses.reshape(B, 1).astype(jnp.int32)
    out = pl.pallas_call(
        _moe_body,
        grid=(V_TILES, E),
        in_specs=[
            pl.BlockSpec((B, 1), lambda v, e: (0, 0)),
            pl.BlockSpec((B, D_MODEL), lambda v, e: (0, 0)),
            pl.BlockSpec((1, HALF, D_MODEL), lambda v, e: (e, 2 * v, 0)),
            pl.BlockSpec((1, HALF, D_MODEL), lambda v, e: (e, 2 * v + 1, 0)),
            pl.BlockSpec((1, 1, VBLK), lambda v, e: (e, 0, v)),
        ],
        out_specs=pl.BlockSpec((B, VBLK), lambda v, e: (0, v)),
        out_shape=jax.ShapeDtypeStruct((B, VOCAB), jnp.float32),
        compiler_params=pltpu.CompilerParams(
            dimension_semantics=("arbitrary", "arbitrary"),
        ),
    )(p2d, x, W, W, b.reshape(E, 1, VOCAB))
    return out
